# Initial kernel scaffold; baseline (speedup 1.0000x reference)
import jax
import jax.numpy as jnp
from jax.experimental import pallas as pl


def kernel(input_ids, W_token, W_number):
    raise NotImplementedError("write your pallas kernel here")


# speedup vs baseline: 1.5143x; 1.5143x over previous
import functools

import jax
import jax.numpy as jnp
from jax import lax
from jax.experimental import pallas as pl
from jax.experimental.pallas import tpu as pltpu
from jax.experimental.pallas import tpu_sc as plsc

VOCAB = 1000000
EMBED_DIM = 32
BATCH = 4096
SEQ = 200

N = BATCH * SEQ
NW = 32
PER_W = N // NW
CHUNK = 1024
SUB = 128
N_SUB = CHUNK // SUB
N_CHUNKS = PER_W // CHUNK
LANES = 16

_mesh = plsc.VectorSubcoreMesh(core_axis_name="c", subcore_axis_name="s")


def _body(ids_hbm, wt_hbm, wn_hbm, out_hbm, idx_v, t_v, n_v, o_v, sem):
    wid = lax.axis_index("s") * 2 + lax.axis_index("c")

    def chunk_step(c, _):
        base = wid * PER_W + c * CHUNK
        pltpu.sync_copy(ids_hbm.at[pl.ds(wid * (PER_W // SUB) + c * N_SUB, N_SUB)],
                        idx_v)
        copies = []
        for j in range(N_SUB):
            copies.append(pltpu.async_copy(
                wt_hbm.at[idx_v.at[j]], t_v.at[pl.ds(j * SUB, SUB)], sem))
            copies.append(pltpu.async_copy(
                wn_hbm.at[idx_v.at[j]], n_v.at[pl.ds(j * SUB, SUB)], sem))
        for cp in copies:
            cp.wait()

        def add_row(r, _):
            for h in range(2):
                sl = pl.ds(h * LANES, LANES)
                o_v[r, sl] = t_v[r, sl] + n_v[r, sl]
            return _

        lax.fori_loop(0, CHUNK, add_row, 0, unroll=4)

        pltpu.sync_copy(o_v, out_hbm.at[pl.ds(base, CHUNK)])
        return _

    lax.fori_loop(0, N_CHUNKS, chunk_step, 0)


_lookup = functools.partial(
    pl.kernel,
    out_type=jax.ShapeDtypeStruct((N, EMBED_DIM), jnp.float32),
    mesh=_mesh,
    scratch_types=[
        pltpu.VMEM((N_SUB, SUB), jnp.int32),
        pltpu.VMEM((CHUNK, EMBED_DIM), jnp.float32),
        pltpu.VMEM((CHUNK, EMBED_DIM), jnp.float32),
        pltpu.VMEM((CHUNK, EMBED_DIM), jnp.float32),
        pltpu.SemaphoreType.DMA,
    ],
    compiler_params=pltpu.CompilerParams(use_tc_tiling_on_sc=False),
)(_body)


@jax.jit
def kernel(input_ids, W_token, W_number):
    ids = input_ids.reshape(-1).astype(jnp.int32).reshape(N // SUB, SUB)
    out = _lookup(ids, W_token, W_number)
    return out.reshape(BATCH, SEQ, EMBED_DIM)


# speedup vs baseline: 1.5687x; 1.0359x over previous
import functools

import jax
import jax.numpy as jnp
from jax import lax
from jax.experimental import pallas as pl
from jax.experimental.pallas import tpu as pltpu
from jax.experimental.pallas import tpu_sc as plsc

VOCAB = 1000000
EMBED_DIM = 32
BATCH = 4096
SEQ = 200

LANES = 16
SUB = 128
N_BT = BATCH // SUB
N_GROUPS = SEQ * N_BT
NW = 32
PER_W = N_GROUPS // NW
G = 4
N_CHUNKS = PER_W // G

_mesh = plsc.VectorSubcoreMesh(core_axis_name="c", subcore_axis_name="s")


def _body(ids_hbm, wt_hbm, wn_hbm, out_hbm, idx_v, t_v, n_v, tr_v, sem, sem_o):
    wid = lax.axis_index("s") * 2 + lax.axis_index("c")
    row0_w = wid * PER_W
    lane_iota = lax.iota(jnp.int32, LANES)

    def chunk_step(c, _):
        row0 = row0_w + c * G
        pltpu.sync_copy(ids_hbm.at[pl.ds(row0, G)], idx_v)
        copies = []
        for g in range(G):
            copies.append(pltpu.async_copy(
                wt_hbm.at[idx_v.at[g]], t_v.at[pl.ds(g * SUB, SUB)], sem))
            copies.append(pltpu.async_copy(
                wn_hbm.at[idx_v.at[g]], n_v.at[pl.ds(g * SUB, SUB)], sem))
        for cp in copies:
            cp.wait()

        for g in range(G):
            def add_row(p, _, g=g):
                for h in range(2):
                    sl = pl.ds(h * LANES, LANES)
                    x = t_v[g * SUB + p, sl] + n_v[g * SUB + p, sl]
                    ridx = lane_iota + (g * EMBED_DIM + h * LANES)
                    cidx = jnp.full((LANES,), p, jnp.int32)
                    plsc.store_scatter(tr_v, [ridx, cidx], x)
                return _

            lax.fori_loop(0, SUB, add_row, 0, unroll=4)

        out_copies = []
        for g in range(G):
            grow = row0 + g
            s = grow // N_BT
            bt = grow % N_BT
            for db in range(4):
                out_copies.append(pltpu.async_copy(
                    tr_v.at[pl.ds(g * EMBED_DIM + db * 8, 8)],
                    out_hbm.at[s, db, bt], sem_o))
        for cp in out_copies:
            cp.wait()
        return _

    lax.fori_loop(0, N_CHUNKS, chunk_step, 0)


_lookup = functools.partial(
    pl.kernel,
    out_type=jax.ShapeDtypeStruct((SEQ, 4, N_BT, 8, SUB), jnp.float32),
    mesh=_mesh,
    scratch_types=[
        pltpu.VMEM((G, SUB), jnp.int32),
        pltpu.VMEM((G * SUB, EMBED_DIM), jnp.float32),
        pltpu.VMEM((G * SUB, EMBED_DIM), jnp.float32),
        pltpu.VMEM((G * EMBED_DIM, SUB), jnp.float32),
        pltpu.SemaphoreType.DMA,
        pltpu.SemaphoreType.DMA,
    ],
    compiler_params=pltpu.CompilerParams(
        use_tc_tiling_on_sc=False, needs_layout_passes=False),
)(_body)


@jax.jit
def kernel(input_ids, W_token, W_number):
    ids = input_ids.astype(jnp.int32).T.reshape(N_GROUPS, SUB)
    out = _lookup(ids, W_token, W_number)
    return out.transpose(2, 4, 0, 1, 3).reshape(BATCH, SEQ, EMBED_DIM)
